# baseline (speedup 1.0000x reference)
import functools

import jax
import jax.numpy as jnp
from jax import lax
from jax.experimental import pallas as pl
from jax.experimental.pallas import tpu as pltpu
from jax.experimental.pallas import tpu_sc as plsc

N = 100000
C = 64
W128 = 64
K27 = 27
TILE = 1024
NTILES = 99
N_PAD = TILE * NTILES
SC_CHUNK = 512
N_CHUNKS = N_PAD // SC_CHUNK
N_WORKERS = 32
CH_PER_W = (N_CHUNKS + N_WORKERS - 1) // N_WORKERS


def _emb_body(emb_ref, we_ref, be_ref, o_ref):
    e = emb_ref[...]
    act = e * jax.nn.sigmoid(e)
    o_ref[...] = jnp.dot(act, we_ref[...], preferred_element_type=jnp.float32) + be_ref[...]


def _k1_body(x_ref, w_ref, g_ref, b_ref, b2_ref, y_ref, fb2_ref, *, n_valid):
    i = pl.program_id(0)
    x = x_ref[...]
    mu = jnp.mean(x, axis=-1, keepdims=True)
    var = jnp.mean((x - mu) ** 2, axis=-1, keepdims=True)
    h = (x - mu) * lax.rsqrt(var + 1e-6) * g_ref[...] + b_ref[...]
    h = h * jax.nn.sigmoid(h)
    rid = i * TILE + lax.broadcasted_iota(jnp.int32, (TILE, C), 0)
    h = jnp.where(rid < n_valid, h, 0.0)
    ybig = jnp.dot(h, w_ref[...], preferred_element_type=jnp.float32)
    for k in range(K27):
        y_ref[k] = ybig[:, k * C:(k + 1) * C]
    fb2_ref[...] = x + b2_ref[...]


def _k3_body(x_ref, b1_ref, sc_ref, sh_ref, bt_ref, w_ref, y_ref, *, n_valid):
    i = pl.program_id(0)
    x = x_ref[...][:, :C] + b1_ref[...]
    mu = jnp.mean(x, axis=-1, keepdims=True)
    var = jnp.mean((x - mu) ** 2, axis=-1, keepdims=True)
    h = (x - mu) * lax.rsqrt(var + 1e-6)
    onehot = (bt_ref[...] == lax.broadcasted_iota(jnp.int32, (TILE, 8), 1)).astype(jnp.float32)
    scale = jnp.dot(onehot, sc_ref[...], preferred_element_type=jnp.float32)
    shift = jnp.dot(onehot, sh_ref[...], preferred_element_type=jnp.float32)
    h = h * (1.0 + scale) + shift
    h = h * jax.nn.sigmoid(h)
    rid = i * TILE + lax.broadcasted_iota(jnp.int32, (TILE, C), 0)
    h = jnp.where(rid < n_valid, h, 0.0)
    ybig = jnp.dot(h, w_ref[...], preferred_element_type=jnp.float32)
    for k in range(K27):
        y_ref[k] = ybig[:, k * C:(k + 1) * C]


def _sc_gather_body(y_hbm, init_hbm, nbr_hbm, out_hbm, idx_v, acc_v):
    wid = lax.axis_index("s") * 2 + lax.axis_index("c")

    def chunk_step(j, carry):
        c = wid + j * N_WORKERS

        @pl.when(c < N_CHUNKS)
        def _():
            base = c * SC_CHUNK
            pltpu.sync_copy(nbr_hbm.at[:, pl.ds(base, SC_CHUNK)], idx_v)
            pltpu.sync_copy(init_hbm.at[pl.ds(base, SC_CHUNK)], acc_v)

            def k_step(k, carry2):
                pltpu.sync_copy(y_hbm.at[k].at[idx_v.at[k]], acc_v, add=True)
                return carry2

            lax.fori_loop(0, K27, k_step, 0)
            pltpu.sync_copy(acc_v, out_hbm.at[pl.ds(base, SC_CHUNK)])

        return carry

    lax.fori_loop(0, CH_PER_W, chunk_step, 0)


def _make_sc_gather():
    return pl.kernel(
        _sc_gather_body,
        out_type=jax.ShapeDtypeStruct((N_PAD, W128), jnp.float32),
        mesh=plsc.VectorSubcoreMesh(
            core_axis_name="c", subcore_axis_name="s", num_cores=2, num_subcores=16
        ),
        compiler_params=pltpu.CompilerParams(use_tc_tiling_on_sc=False),
        scratch_types=[
            pltpu.VMEM((K27, SC_CHUNK), jnp.int32),
            pltpu.VMEM((SC_CHUNK, W128), jnp.float32),
        ],
    )


def kernel(feats, emb, gamma1, beta1, W1, b1, W2, b2, We, be, nbr_idx, batch_idx, num_frames):
    f32 = jnp.float32
    feats = feats.astype(f32)
    pad = N_PAD - N
    feats_pad = jnp.concatenate([feats, jnp.zeros((pad, C), f32)], axis=0)
    w1cat = jnp.transpose(W1.astype(f32), (1, 0, 2)).reshape(C, K27 * C)
    w2cat = jnp.transpose(W2.astype(f32), (1, 0, 2)).reshape(C, K27 * C)
    nbrT = jnp.asarray(nbr_idx, jnp.int32).T
    spread = N + (lax.broadcasted_iota(jnp.int32, nbrT.shape, 1) % pad)
    nbrT = jnp.where(nbrT == N, spread, nbrT)
    nbrT = jnp.concatenate(
        [nbrT, jnp.broadcast_to(jnp.arange(N, N + pad, dtype=jnp.int32), (K27, pad))],
        axis=1,
    )
    batch_pad = jnp.concatenate(
        [jnp.asarray(batch_idx, jnp.int32), jnp.zeros((pad,), jnp.int32)]
    ).reshape(N_PAD, 1)
    emb8 = jnp.zeros((8, emb.shape[1]), f32).at[:4].set(emb.astype(f32))
    be8 = jnp.broadcast_to(be.astype(f32).reshape(1, -1), (8, 2 * C))

    emb_out = pl.pallas_call(
        _emb_body,
        out_shape=jax.ShapeDtypeStruct((8, 2 * C), f32),
    )(emb8, We.astype(f32), be8)
    scale8 = emb_out[:, :C]
    shift8 = emb_out[:, C:]

    y1, fb2 = pl.pallas_call(
        functools.partial(_k1_body, n_valid=N),
        grid=(NTILES,),
        in_specs=[
            pl.BlockSpec((TILE, C), lambda i: (i, 0)),
            pl.BlockSpec((C, K27 * C), lambda i: (0, 0)),
            pl.BlockSpec((1, C), lambda i: (0, 0)),
            pl.BlockSpec((1, C), lambda i: (0, 0)),
            pl.BlockSpec((1, C), lambda i: (0, 0)),
        ],
        out_specs=[
            pl.BlockSpec((K27, TILE, W128), lambda i: (0, i, 0)),
            pl.BlockSpec((TILE, W128), lambda i: (i, 0)),
        ],
        out_shape=[
            jax.ShapeDtypeStruct((K27, N_PAD, W128), f32),
            jax.ShapeDtypeStruct((N_PAD, W128), f32),
        ],
    )(feats_pad, w1cat, gamma1.astype(f32).reshape(1, C),
      beta1.astype(f32).reshape(1, C), b2.astype(f32).reshape(1, C))

    zeros_tab = jnp.zeros((N_PAD, W128), f32)
    out1 = _make_sc_gather()(y1, zeros_tab, nbrT)

    y2 = pl.pallas_call(
        functools.partial(_k3_body, n_valid=N),
        grid=(NTILES,),
        in_specs=[
            pl.BlockSpec((TILE, W128), lambda i: (i, 0)),
            pl.BlockSpec((1, C), lambda i: (0, 0)),
            pl.BlockSpec((8, C), lambda i: (0, 0)),
            pl.BlockSpec((8, C), lambda i: (0, 0)),
            pl.BlockSpec((TILE, 1), lambda i: (i, 0)),
            pl.BlockSpec((C, K27 * C), lambda i: (0, 0)),
        ],
        out_specs=pl.BlockSpec((K27, TILE, W128), lambda i: (0, i, 0)),
        out_shape=jax.ShapeDtypeStruct((K27, N_PAD, W128), f32),
    )(out1, b1.astype(f32).reshape(1, C), scale8, shift8, batch_pad, w2cat)

    out = _make_sc_gather()(y2, fb2, nbrT)
    return out[:N, :C] if W128 != C else out[:N]


# speedup vs baseline: 1.7151x; 1.7151x over previous
import functools

import jax
import jax.numpy as jnp
from jax import lax
from jax.experimental import pallas as pl
from jax.experimental.pallas import tpu as pltpu
from jax.experimental.pallas import tpu_sc as plsc

N = 100000
NB = 4
NPB = N // NB
C = 64
W128 = 128
K27 = 27
TILE = 512
NTILES = 50
BP = TILE * NTILES
SC_CHUNK = 128
N_CHUNKS = BP // SC_CHUNK
N_WORKERS = 32
CH_PER_W = (N_CHUNKS + N_WORKERS - 1) // N_WORKERS


def _emb_body(emb_ref, we_ref, be_ref, o_ref):
    e = emb_ref[...]
    act = e * jax.nn.sigmoid(e)
    o_ref[...] = jnp.dot(act, we_ref[...], preferred_element_type=jnp.float32) + be_ref[...]


def _k1_body(x_ref, w_ref, g_ref, b_ref, b2_ref, y_ref, fb2_ref, *, n_valid):
    i = pl.program_id(0)
    x = x_ref[...]
    mu = jnp.mean(x, axis=-1, keepdims=True)
    var = jnp.mean((x - mu) ** 2, axis=-1, keepdims=True)
    h = (x - mu) * lax.rsqrt(var + 1e-6) * g_ref[...] + b_ref[...]
    h = h * jax.nn.sigmoid(h)
    rid = i * TILE + lax.broadcasted_iota(jnp.int32, (TILE, C), 0)
    h = jnp.where(rid < n_valid, h, 0.0)
    ybig = jnp.dot(h, w_ref[...], preferred_element_type=jnp.float32)
    zpad = jnp.zeros((TILE, W128 - C), jnp.float32)
    for k in range(K27):
        y_ref[k] = jnp.concatenate([ybig[:, k * C:(k + 1) * C], zpad], axis=1)
    fb2_ref[...] = jnp.concatenate([x + b2_ref[...], zpad], axis=1)


def _k3_body(x_ref, b1_ref, sc_ref, sh_ref, w_ref, y_ref, *, n_valid):
    i = pl.program_id(0)
    x = x_ref[...][:, :C] + b1_ref[...]
    mu = jnp.mean(x, axis=-1, keepdims=True)
    var = jnp.mean((x - mu) ** 2, axis=-1, keepdims=True)
    h = (x - mu) * lax.rsqrt(var + 1e-6)
    h = h * (1.0 + sc_ref[...]) + sh_ref[...]
    h = h * jax.nn.sigmoid(h)
    rid = i * TILE + lax.broadcasted_iota(jnp.int32, (TILE, C), 0)
    h = jnp.where(rid < n_valid, h, 0.0)
    ybig = jnp.dot(h, w_ref[...], preferred_element_type=jnp.float32)
    zpad = jnp.zeros((TILE, W128 - C), jnp.float32)
    for k in range(K27):
        y_ref[k] = jnp.concatenate([ybig[:, k * C:(k + 1) * C], zpad], axis=1)


def _sc_gather_body(y_hbm, init_hbm, nbr_hbm, out_hbm, idx_v, acc_v, sem, *, use_init):
    wid = lax.axis_index("s") * 2 + lax.axis_index("c")

    def chunk_step(j, carry):
        c = wid + j * N_WORKERS

        @pl.when(c < N_CHUNKS)
        def _():
            base = c * SC_CHUNK
            pltpu.sync_copy(nbr_hbm.at[:, pl.ds(base, SC_CHUNK)], idx_v)
            if use_init:
                pltpu.sync_copy(init_hbm.at[pl.ds(base, SC_CHUNK)], acc_v)
                pltpu.async_copy(y_hbm.at[0].at[idx_v.at[0]], acc_v, sem, add=True)
            else:
                pltpu.async_copy(y_hbm.at[0].at[idx_v.at[0]], acc_v, sem).wait()

            def fire(k, carry2):
                pltpu.async_copy(y_hbm.at[k].at[idx_v.at[k]], acc_v, sem, add=True)
                return carry2

            lax.fori_loop(1, K27, fire, 0)
            n_drain = K27 if use_init else K27 - 1

            def drain(k, carry2):
                pltpu.make_async_copy(y_hbm.at[0].at[idx_v.at[0]], acc_v, sem).wait()
                return carry2

            lax.fori_loop(0, n_drain, drain, 0)
            pltpu.sync_copy(acc_v, out_hbm.at[pl.ds(base, SC_CHUNK)])

        return carry

    lax.fori_loop(0, CH_PER_W, chunk_step, 0)


def _make_sc_gather(use_init):
    return pl.kernel(
        functools.partial(_sc_gather_body, use_init=use_init),
        out_type=jax.ShapeDtypeStruct((BP, W128), jnp.float32),
        mesh=plsc.VectorSubcoreMesh(
            core_axis_name="c", subcore_axis_name="s", num_cores=2, num_subcores=16
        ),
        compiler_params=pltpu.CompilerParams(use_tc_tiling_on_sc=False),
        scratch_types=[
            pltpu.VMEM((K27, SC_CHUNK), jnp.int32),
            pltpu.VMEM((SC_CHUNK, W128), jnp.float32),
            pltpu.SemaphoreType.DMA,
        ],
    )


def kernel(feats, emb, gamma1, beta1, W1, b1, W2, b2, We, be, nbr_idx, batch_idx, num_frames):
    f32 = jnp.float32
    feats = feats.astype(f32)
    pad = BP - NPB
    w1cat = jnp.transpose(W1.astype(f32), (1, 0, 2)).reshape(C, K27 * C)
    w2cat = jnp.transpose(W2.astype(f32), (1, 0, 2)).reshape(C, K27 * C)
    nbrT_g = jnp.asarray(nbr_idx, jnp.int32).T
    emb8 = jnp.zeros((8, emb.shape[1]), f32).at[:4].set(emb.astype(f32))
    be8 = jnp.broadcast_to(be.astype(f32).reshape(1, -1), (8, 2 * C))

    emb_out = pl.pallas_call(
        _emb_body,
        out_shape=jax.ShapeDtypeStruct((8, 2 * C), f32),
    )(emb8, We.astype(f32), be8)

    gam = gamma1.astype(f32).reshape(1, C)
    bet = beta1.astype(f32).reshape(1, C)
    b1r = b1.astype(f32).reshape(1, C)
    b2r = b2.astype(f32).reshape(1, C)

    k1_fn = pl.pallas_call(
        functools.partial(_k1_body, n_valid=NPB),
        grid=(NTILES,),
        in_specs=[
            pl.BlockSpec((TILE, C), lambda i: (i, 0)),
            pl.BlockSpec((C, K27 * C), lambda i: (0, 0)),
            pl.BlockSpec((1, C), lambda i: (0, 0)),
            pl.BlockSpec((1, C), lambda i: (0, 0)),
            pl.BlockSpec((1, C), lambda i: (0, 0)),
        ],
        out_specs=[
            pl.BlockSpec((K27, TILE, W128), lambda i: (0, i, 0)),
            pl.BlockSpec((TILE, W128), lambda i: (i, 0)),
        ],
        out_shape=[
            jax.ShapeDtypeStruct((K27, BP, W128), f32),
            jax.ShapeDtypeStruct((BP, W128), f32),
        ],
    )
    k3_fn = pl.pallas_call(
        functools.partial(_k3_body, n_valid=NPB),
        grid=(NTILES,),
        in_specs=[
            pl.BlockSpec((TILE, W128), lambda i: (i, 0)),
            pl.BlockSpec((1, C), lambda i: (0, 0)),
            pl.BlockSpec((1, C), lambda i: (0, 0)),
            pl.BlockSpec((1, C), lambda i: (0, 0)),
            pl.BlockSpec((C, K27 * C), lambda i: (0, 0)),
        ],
        out_specs=pl.BlockSpec((K27, TILE, W128), lambda i: (0, i, 0)),
        out_shape=jax.ShapeDtypeStruct((K27, BP, W128), f32),
    )

    outs = []
    for b in range(NB):
        lo = b * NPB
        feats_b = jnp.concatenate(
            [lax.slice_in_dim(feats, lo, lo + NPB, axis=0), jnp.zeros((pad, C), f32)],
            axis=0,
        )
        nbr_b = lax.slice_in_dim(nbrT_g, lo, lo + NPB, axis=1) - lo
        spread = NPB + (lax.broadcasted_iota(jnp.int32, nbr_b.shape, 1) % pad)
        nbr_b = jnp.where(nbr_b == N - lo, spread, nbr_b)
        nbr_b = jnp.concatenate(
            [nbr_b, jnp.broadcast_to(jnp.arange(NPB, BP, dtype=jnp.int32), (K27, pad))],
            axis=1,
        )
        scale_b = lax.slice(emb_out, (b, 0), (b + 1, C))
        shift_b = lax.slice(emb_out, (b, C), (b + 1, 2 * C))

        y1, fb2 = k1_fn(feats_b, w1cat, gam, bet, b2r)
        out1 = _make_sc_gather(False)(y1, fb2, nbr_b)
        y2 = k3_fn(out1, b1r, scale_b, shift_b, w2cat)
        outs.append(_make_sc_gather(True)(y2, fb2, nbr_b)[:NPB, :C])
    return jnp.concatenate(outs, axis=0)
